# baseline (speedup 1.0000x reference)
import functools

import jax
import jax.numpy as jnp
from jax import lax
from jax.experimental import pallas as pl
from jax.experimental.pallas import tpu as pltpu
from jax.experimental.pallas import tpu_sc as plsc

B = 16384
L = 50
V = 100000
R = 5
D = 64

NC = 2
NS = 16
NW = NC * NS
BW = B // NW
PAIR = 2
PROWS = PAIR * L
PPAD = 104
NPAIR_W = BW // PAIR
NBUF = 2

_f32 = jnp.float32


def _stage1_body(e_ref, f_ref, wt_ref, rr_ref, wb_ref, bg_ref, t_ref, c_ref):
    t_ref[:, :D] = jnp.dot(e_ref[...], wt_ref[...],
                           preferred_element_type=_f32)
    t_ref[:, D:] = f_ref[...]

    @pl.when(pl.program_id(0) == 0)
    def _():
        c_ref[:, :D] = jnp.dot(rr_ref[...], wb_ref[...],
                               preferred_element_type=_f32) + bg_ref[...]


def _stage1(e_table, feat_table, wt, rr, wb, bg):
    rows = 800
    grid = V // rows
    return pl.pallas_call(
        _stage1_body,
        grid=(grid,),
        in_specs=[
            pl.BlockSpec((rows, D), lambda i: (i, 0)),
            pl.BlockSpec((rows, D), lambda i: (i, 0)),
            pl.BlockSpec((D, D), lambda i: (0, 0)),
            pl.BlockSpec((32, D), lambda i: (0, 0)),
            pl.BlockSpec((D, D), lambda i: (0, 0)),
            pl.BlockSpec((1, D), lambda i: (0, 0)),
        ],
        out_specs=[
            pl.BlockSpec((rows, 2 * D), lambda i: (i, 0)),
            pl.BlockSpec((32, 2 * D), lambda i: (0, 0)),
        ],
        out_shape=[
            jax.ShapeDtypeStruct((V, 2 * D), _f32),
            jax.ShapeDtypeStruct((32, 2 * D), _f32),
        ],
    )(e_table, feat_table, wt, rr, wb, bg)


def _stage2_body(t_hbm, uvp_hbm, cidx_hbm, nodes_hbm, c_hbm,
                 neigh_hbm, selff_hbm,
                 uvp_v, cidx_v, c_v, nodes_v, out_v, rows_v,
                 sem0, sem1):
    wid = lax.axis_index("s") * NC + lax.axis_index("c")
    base = wid * BW
    sems = (sem0, sem1)

    pltpu.sync_copy(c_hbm, c_v)
    pltpu.sync_copy(nodes_hbm.at[pl.ds(base, BW)], nodes_v)
    pltpu.sync_copy(uvp_hbm.at[pl.ds(wid * NPAIR_W, NPAIR_W)], uvp_v)
    pltpu.sync_copy(cidx_hbm.at[pl.ds(base * L, BW * L)],
                    cidx_v.at[pl.ds(0, BW * L)])

    for q in range(BW // 64):
        pltpu.async_copy(t_hbm.at[nodes_v.at[pl.ds(q * 64, 64)]],
                         out_v, sem0).wait()
        pltpu.sync_copy(out_v, selff_hbm.at[pl.ds(base + q * 64, 64)])

    def start(u, pair):
        pltpu.make_async_copy(
            t_hbm.at[uvp_v.at[pair]], rows_v.at[u], sems[u]).start()

    def wait(u):
        pltpu.make_async_copy(
            t_hbm.at[uvp_v.at[0]], rows_v.at[u], sems[u]).wait()

    inv_l = _f32(1.0 / L)
    zero = jnp.zeros((16,), _f32)

    def compute(u, pair):
        for bb in range(PAIR):
            lb = (pair % 32) * PAIR + bb
            cbase = pair * PROWS + bb * L
            accs = [zero, zero, zero, zero]
            for i in range(L):
                ci = cidx_v[pl.ds(cbase + i, 16)][0]
                row = bb * L + i
                for j in range(4):
                    y = rows_v[u, row, pl.ds(j * 16, 16)]
                    c = c_v[ci, pl.ds(j * 16, 16)]
                    accs[j] = accs[j] + jnp.maximum(y + c, 0.0)
            for j in range(4):
                out_v[lb, pl.ds(j * 16, 16)] = accs[j] * inv_l

    for u in range(NBUF):
        start(u, u)

    def outer(t, _):
        p = t * NBUF
        for u in range(NBUF):
            pair = p + u
            wait(u)
            compute(u, pair)

            @pl.when(pair + NBUF < NPAIR_W)
            def _():
                start(u, pair + NBUF)

            @pl.when(pair % 32 == 31)
            def _():
                pltpu.sync_copy(
                    out_v,
                    neigh_hbm.at[pl.ds(base + (pair // 32) * 64, 64)])
        return 0

    lax.fori_loop(0, NPAIR_W // NBUF, outer, 0)


def _stage2(t, uvp, cidx, nodes, c):
    mesh = plsc.VectorSubcoreMesh(core_axis_name="c", subcore_axis_name="s")
    kern = functools.partial(
        pl.kernel,
        mesh=mesh,
        out_type=[
            jax.ShapeDtypeStruct((B, 2 * D), _f32),
            jax.ShapeDtypeStruct((B, 2 * D), _f32),
        ],
        scratch_types=[
            pltpu.VMEM((NPAIR_W, PPAD), jnp.int32),
            pltpu.VMEM((BW * L + 16,), jnp.int32),
            pltpu.VMEM((32, 2 * D), _f32),
            pltpu.VMEM((BW,), jnp.int32),
            pltpu.VMEM((64, 2 * D), _f32),
            pltpu.VMEM((NBUF, PPAD, 2 * D), _f32),
            pltpu.SemaphoreType.DMA,
            pltpu.SemaphoreType.DMA,
        ],
    )(_stage2_body)
    return kern(t, uvp, cidx, nodes, c)


def _stage3_body(sf_ref, ng_ref, w1t_ref, w1b_ref, b1_ref, o_ref):
    sf = sf_ref[:, D:]
    ng = ng_ref[:, :D]
    o_ref[...] = jnp.maximum(
        jnp.dot(sf, w1t_ref[...], preferred_element_type=_f32)
        + jnp.dot(ng, w1b_ref[...], preferred_element_type=_f32)
        + b1_ref[...], 0.0)


def _stage3(selff, neigh, w1t, w1b, b1):
    rows = 1024
    return pl.pallas_call(
        _stage3_body,
        grid=(B // rows,),
        in_specs=[
            pl.BlockSpec((rows, 2 * D), lambda i: (i, 0)),
            pl.BlockSpec((rows, 2 * D), lambda i: (i, 0)),
            pl.BlockSpec((D, D), lambda i: (0, 0)),
            pl.BlockSpec((D, D), lambda i: (0, 0)),
            pl.BlockSpec((1, D), lambda i: (0, 0)),
        ],
        out_specs=pl.BlockSpec((rows, D), lambda i: (i, 0)),
        out_shape=jax.ShapeDtypeStruct((B, D), _f32),
    )(selff, neigh, w1t, w1b, b1)


def kernel(nodes, history_uv, history_ra, history_re, feat_table, e_table,
           r_table, re_table, W_gv, b_gv, W1, b1):
    wt = W_gv[:D]
    wb = W_gv[D:]
    rr = (r_table[:, None, :] + re_table[None, :, :]).reshape(R * R, D)
    rr = jnp.pad(rr, ((0, 32 - R * R), (0, 0)))
    t, c = _stage1(e_table, feat_table, wt, rr, wb, b_gv.reshape(1, D))

    uvp = jnp.pad(history_uv.astype(jnp.int32).reshape(B // PAIR, PROWS),
                  ((0, 0), (0, PPAD - PROWS)))
    cidx = (history_ra * R + history_re).astype(jnp.int32).reshape(-1)
    neigh, selff = _stage2(t, uvp, cidx, nodes.astype(jnp.int32), c)

    return _stage3(selff, neigh, W1[:D], W1[D:], b1.reshape(1, D))


# speedup vs baseline: 2.3536x; 2.3536x over previous
import functools

import jax
import jax.numpy as jnp
from jax import lax
from jax.experimental import pallas as pl
from jax.experimental.pallas import tpu as pltpu
from jax.experimental.pallas import tpu_sc as plsc

B = 16384
L = 50
V = 100000
R = 5
D = 64

NC = 2
NS = 16
NW = NC * NS
BW = B // NW
PAIR = 2
PROWS = PAIR * L
PPAD = 104
NPAIR_W = BW // PAIR
NBUF = 4

_f32 = jnp.float32


def _stage1_body(e_ref, wt_ref, rr_ref, wb_ref, bg_ref, y_ref, c_ref):
    y_ref[...] = jnp.dot(e_ref[...], wt_ref[...],
                         preferred_element_type=_f32).astype(jnp.bfloat16)

    @pl.when(pl.program_id(0) == 0)
    def _():
        c_ref[...] = (jnp.dot(rr_ref[...], wb_ref[...],
                              preferred_element_type=_f32)
                      + bg_ref[...]).astype(jnp.bfloat16)


def _stage1(e_table, wt, rr, wb, bg):
    rows = 800
    grid = V // rows
    return pl.pallas_call(
        _stage1_body,
        grid=(grid,),
        in_specs=[
            pl.BlockSpec((rows, D), lambda i: (i, 0)),
            pl.BlockSpec((D, D), lambda i: (0, 0)),
            pl.BlockSpec((32, D), lambda i: (0, 0)),
            pl.BlockSpec((D, D), lambda i: (0, 0)),
            pl.BlockSpec((1, D), lambda i: (0, 0)),
        ],
        out_specs=[
            pl.BlockSpec((rows, D), lambda i: (i, 0)),
            pl.BlockSpec((32, D), lambda i: (0, 0)),
        ],
        out_shape=[
            jax.ShapeDtypeStruct((V, D), jnp.bfloat16),
            jax.ShapeDtypeStruct((32, D), jnp.bfloat16),
        ],
    )(e_table, wt, rr, wb, bg)


def _stage2_body(y_hbm, uvp_hbm, cidx_hbm, nodes_hbm, feat_hbm, c_hbm,
                 neigh_hbm, selff_hbm,
                 uvp_v, cidx_v, c_v, nodes_v, out_v, rows_v,
                 sem0, sem1, sem2, sem3):
    wid = lax.axis_index("s") * NC + lax.axis_index("c")
    base = wid * BW
    sems = (sem0, sem1, sem2, sem3)

    pltpu.sync_copy(c_hbm, c_v)
    pltpu.sync_copy(nodes_hbm.at[pl.ds(base, BW)], nodes_v)
    pltpu.sync_copy(uvp_hbm.at[pl.ds(wid * NPAIR_W, NPAIR_W)], uvp_v)
    pltpu.sync_copy(cidx_hbm.at[pl.ds(base, BW)], cidx_v)

    for q in range(BW // 64):
        pltpu.async_copy(feat_hbm.at[nodes_v.at[pl.ds(q * 64, 64)]],
                         out_v, sem0).wait()
        pltpu.sync_copy(out_v, selff_hbm.at[pl.ds(base + q * 64, 64)])

    def start(u, pair):
        pltpu.make_async_copy(
            y_hbm.at[uvp_v.at[pair]], rows_v.at[u], sems[u]).start()

    def wait(u):
        pltpu.make_async_copy(
            y_hbm.at[uvp_v.at[0]], rows_v.at[u], sems[u]).wait()

    inv_l = _f32(1.0 / L)
    zero = jnp.zeros((16,), _f32)

    def compute(u, pair):
        for bb in range(PAIR):
            lb = (pair % 32) * PAIR + bb
            accs = [zero, zero, zero, zero]
            for g in range((L + 15) // 16):
                civ = cidx_v[pair * PAIR + bb, pl.ds(g * 16, 16)]
                for k in range(min(16, L - g * 16)):
                    ci = civ[k]
                    row = bb * L + g * 16 + k
                    for h in range(2):
                        y = rows_v[u, row, pl.ds(h * 32, 32)]
                        c = c_v[ci, pl.ds(h * 32, 32)]
                        t = jnp.maximum(y + c, jnp.bfloat16(0.0))
                        lo, hi = plsc.unpack(
                            t, format=plsc.PackFormat.INTERLEAVED)
                        accs[2 * h] = accs[2 * h] + lo
                        accs[2 * h + 1] = accs[2 * h + 1] + hi
            for j in range(4):
                out_v[lb, pl.ds(j * 16, 16)] = accs[j] * inv_l

    for u in range(NBUF):
        start(u, u)

    def outer(t, _):
        p = t * NBUF
        for u in range(NBUF):
            pair = p + u
            wait(u)
            compute(u, pair)

            @pl.when(pair + NBUF < NPAIR_W)
            def _():
                start(u, pair + NBUF)

            @pl.when(pair % 32 == 31)
            def _():
                pltpu.sync_copy(
                    out_v,
                    neigh_hbm.at[pl.ds(base + (pair // 32) * 64, 64)])
        return 0

    lax.fori_loop(0, NPAIR_W // NBUF, outer, 0)


def _stage2(y, uvp, cidx, nodes, feat_table, c):
    mesh = plsc.VectorSubcoreMesh(core_axis_name="c", subcore_axis_name="s")
    kern = functools.partial(
        pl.kernel,
        mesh=mesh,
        compiler_params=pltpu.CompilerParams(use_tc_tiling_on_sc=False,
                                             needs_layout_passes=False),
        out_type=[
            jax.ShapeDtypeStruct((B, D), _f32),
            jax.ShapeDtypeStruct((B, D), _f32),
        ],
        scratch_types=[
            pltpu.VMEM((NPAIR_W, PPAD), jnp.int32),
            pltpu.VMEM((BW, 64), jnp.int32),
            pltpu.VMEM((32, D), jnp.bfloat16),
            pltpu.VMEM((BW,), jnp.int32),
            pltpu.VMEM((64, D), _f32),
            pltpu.VMEM((NBUF, PPAD, D), jnp.bfloat16),
            pltpu.SemaphoreType.DMA,
            pltpu.SemaphoreType.DMA,
            pltpu.SemaphoreType.DMA,
            pltpu.SemaphoreType.DMA,
        ],
    )(_stage2_body)
    return kern(y, uvp, cidx, nodes, feat_table, c)


def _stage3_body(sf_ref, ng_ref, w1t_ref, w1b_ref, b1_ref, o_ref):
    o_ref[...] = jnp.maximum(
        jnp.dot(sf_ref[...], w1t_ref[...], preferred_element_type=_f32)
        + jnp.dot(ng_ref[...], w1b_ref[...], preferred_element_type=_f32)
        + b1_ref[...], 0.0)


def _stage3(selff, neigh, w1t, w1b, b1):
    rows = 1024
    return pl.pallas_call(
        _stage3_body,
        grid=(B // rows,),
        in_specs=[
            pl.BlockSpec((rows, D), lambda i: (i, 0)),
            pl.BlockSpec((rows, D), lambda i: (i, 0)),
            pl.BlockSpec((D, D), lambda i: (0, 0)),
            pl.BlockSpec((D, D), lambda i: (0, 0)),
            pl.BlockSpec((1, D), lambda i: (0, 0)),
        ],
        out_specs=pl.BlockSpec((rows, D), lambda i: (i, 0)),
        out_shape=jax.ShapeDtypeStruct((B, D), _f32),
    )(selff, neigh, w1t, w1b, b1)


def kernel(nodes, history_uv, history_ra, history_re, feat_table, e_table,
           r_table, re_table, W_gv, b_gv, W1, b1):
    wt = W_gv[:D]
    wb = W_gv[D:]
    rr = (r_table[:, None, :] + re_table[None, :, :]).reshape(R * R, D)
    rr = jnp.pad(rr, ((0, 32 - R * R), (0, 0)))
    y, c = _stage1(e_table, wt, rr, wb, b_gv.reshape(1, D))

    uvp = jnp.pad(history_uv.astype(jnp.int32).reshape(B // PAIR, PROWS),
                  ((0, 0), (0, PPAD - PROWS)))
    cidx = jnp.pad((history_ra * R + history_re).astype(jnp.int32),
                   ((0, 0), (0, 64 - L)))
    neigh, selff = _stage2(y, uvp, cidx, nodes.astype(jnp.int32),
                           feat_table, c)

    perm = jnp.concatenate([
        jnp.arange(0, 32, 2), jnp.arange(1, 32, 2),
        jnp.arange(32, 64, 2), jnp.arange(33, 64, 2)])
    w1b = W1[D:][perm, :]
    return _stage3(selff, neigh, W1[:D], w1b, b1.reshape(1, D))


# speedup vs baseline: 3.0474x; 1.2948x over previous
import functools

import jax
import jax.numpy as jnp
from jax import lax
from jax.experimental import pallas as pl
from jax.experimental.pallas import tpu as pltpu
from jax.experimental.pallas import tpu_sc as plsc

B = 16384
L = 50
V = 100000
R = 5
D = 64

NC = 2
NS = 16
NW = NC * NS
BW = B // NW
PAIR = 2
PROWS = PAIR * L
PPAD = 104
NPAIR_W = BW // PAIR
NBUF = 4

_f32 = jnp.float32


def _stage1_body(e_ref, wt_ref, rr_ref, wb_ref, bg_ref, y_ref, c_ref):
    y_ref[...] = jnp.dot(e_ref[...], wt_ref[...],
                         preferred_element_type=_f32).astype(jnp.bfloat16)

    @pl.when(pl.program_id(0) == 0)
    def _():
        c_ref[...] = (jnp.dot(rr_ref[...], wb_ref[...],
                              preferred_element_type=_f32)
                      + bg_ref[...]).astype(jnp.bfloat16)


def _stage1(e_table, wt, rr, wb, bg):
    rows = 800
    grid = V // rows
    return pl.pallas_call(
        _stage1_body,
        grid=(grid,),
        in_specs=[
            pl.BlockSpec((rows, D), lambda i: (i, 0)),
            pl.BlockSpec((D, D), lambda i: (0, 0)),
            pl.BlockSpec((32, D), lambda i: (0, 0)),
            pl.BlockSpec((D, D), lambda i: (0, 0)),
            pl.BlockSpec((1, D), lambda i: (0, 0)),
        ],
        out_specs=[
            pl.BlockSpec((rows, D), lambda i: (i, 0)),
            pl.BlockSpec((32, D), lambda i: (0, 0)),
        ],
        out_shape=[
            jax.ShapeDtypeStruct((V, D), jnp.bfloat16),
            jax.ShapeDtypeStruct((32, D), jnp.bfloat16),
        ],
    )(e_table, wt, rr, wb, bg)


def _stage2_body(y_hbm, uv_hbm, ra_hbm, re_hbm, nodes_hbm, feat_hbm, c_hbm,
                 neigh_hbm, selff_hbm,
                 uvp_v, ra_v, re_v, c_v, nodes_v, out_v, rows_v,
                 sem0, sem1, sem2, sem3):
    wid = lax.axis_index("s") * NC + lax.axis_index("c")
    base = wid * BW
    sems = (sem0, sem1, sem2, sem3)

    pltpu.sync_copy(c_hbm, c_v)
    pltpu.sync_copy(nodes_hbm.at[pl.ds(base, BW)], nodes_v)
    pltpu.sync_copy(uv_hbm.at[pl.ds(wid * NPAIR_W, NPAIR_W), :], uvp_v)
    pltpu.sync_copy(ra_hbm.at[pl.ds(base * L, BW * L)],
                    ra_v.at[pl.ds(0, BW * L)])
    pltpu.sync_copy(re_hbm.at[pl.ds(base * L, BW * L)],
                    re_v.at[pl.ds(0, BW * L)])

    for q in range(BW // 64):
        pltpu.async_copy(feat_hbm.at[nodes_v.at[pl.ds(q * 64, 64)]],
                         out_v, sem0).wait()
        pltpu.sync_copy(out_v, selff_hbm.at[pl.ds(base + q * 64, 64)])

    def start(u, pair):
        pltpu.make_async_copy(
            y_hbm.at[uvp_v.at[pair]], rows_v.at[u], sems[u]).start()

    def wait(u):
        pltpu.make_async_copy(
            y_hbm.at[uvp_v.at[0]], rows_v.at[u], sems[u]).wait()

    inv_l = _f32(1.0 / L)
    zero = jnp.zeros((16,), _f32)

    def compute(u, pair):
        for bb in range(PAIR):
            lb = (pair % 32) * PAIR + bb
            cbase = pair * PROWS + bb * L
            accs = [zero, zero, zero, zero]
            for g in range((L + 15) // 16):
                rav = ra_v[pl.ds(cbase + g * 16, 16)]
                rev = re_v[pl.ds(cbase + g * 16, 16)]
                civ = rav * R + rev
                for k in range(min(16, L - g * 16)):
                    ci = civ[k]
                    row = bb * L + g * 16 + k
                    for h in range(2):
                        y = rows_v[u, row, pl.ds(h * 32, 32)]
                        c = c_v[ci, pl.ds(h * 32, 32)]
                        t = jnp.maximum(y + c, jnp.bfloat16(0.0))
                        lo, hi = plsc.unpack(
                            t, format=plsc.PackFormat.INTERLEAVED)
                        accs[2 * h] = accs[2 * h] + lo
                        accs[2 * h + 1] = accs[2 * h + 1] + hi
            for j in range(4):
                out_v[lb, pl.ds(j * 16, 16)] = accs[j] * inv_l

    for u in range(NBUF):
        start(u, u)

    def outer(t, _):
        p = t * NBUF
        for u in range(NBUF):
            pair = p + u
            wait(u)
            compute(u, pair)

            @pl.when(pair + NBUF < NPAIR_W)
            def _():
                start(u, pair + NBUF)

            @pl.when(pair % 32 == 31)
            def _():
                pltpu.sync_copy(
                    out_v,
                    neigh_hbm.at[pl.ds(base + (pair // 32) * 64, 64)])
        return 0

    lax.fori_loop(0, NPAIR_W // NBUF, outer, 0)


def _stage2(y, uv2, ra, re, nodes, feat_table, c):
    mesh = plsc.VectorSubcoreMesh(core_axis_name="c", subcore_axis_name="s")
    kern = functools.partial(
        pl.kernel,
        mesh=mesh,
        compiler_params=pltpu.CompilerParams(use_tc_tiling_on_sc=False,
                                             needs_layout_passes=False),
        out_type=[
            jax.ShapeDtypeStruct((B, D), _f32),
            jax.ShapeDtypeStruct((B, D), _f32),
        ],
        scratch_types=[
            pltpu.VMEM((NPAIR_W, PROWS), jnp.int32),
            pltpu.VMEM((BW * L + 16,), jnp.int32),
            pltpu.VMEM((BW * L + 16,), jnp.int32),
            pltpu.VMEM((32, D), jnp.bfloat16),
            pltpu.VMEM((BW,), jnp.int32),
            pltpu.VMEM((64, D), _f32),
            pltpu.VMEM((NBUF, PROWS, D), jnp.bfloat16),
            pltpu.SemaphoreType.DMA,
            pltpu.SemaphoreType.DMA,
            pltpu.SemaphoreType.DMA,
            pltpu.SemaphoreType.DMA,
        ],
    )(_stage2_body)
    return kern(y, uv2, ra, re, nodes, feat_table, c)


def _stage3_body(sf_ref, ng_ref, w1t_ref, w1b_ref, b1_ref, o_ref):
    o_ref[...] = jnp.maximum(
        jnp.dot(sf_ref[...], w1t_ref[...], preferred_element_type=_f32)
        + jnp.dot(ng_ref[...], w1b_ref[...], preferred_element_type=_f32)
        + b1_ref[...], 0.0)


def _stage3(selff, neigh, w1t, w1b, b1):
    rows = 1024
    return pl.pallas_call(
        _stage3_body,
        grid=(B // rows,),
        in_specs=[
            pl.BlockSpec((rows, D), lambda i: (i, 0)),
            pl.BlockSpec((rows, D), lambda i: (i, 0)),
            pl.BlockSpec((D, D), lambda i: (0, 0)),
            pl.BlockSpec((D, D), lambda i: (0, 0)),
            pl.BlockSpec((1, D), lambda i: (0, 0)),
        ],
        out_specs=pl.BlockSpec((rows, D), lambda i: (i, 0)),
        out_shape=jax.ShapeDtypeStruct((B, D), _f32),
    )(selff, neigh, w1t, w1b, b1)


def kernel(nodes, history_uv, history_ra, history_re, feat_table, e_table,
           r_table, re_table, W_gv, b_gv, W1, b1):
    wt = W_gv[:D]
    wb = W_gv[D:]
    rr = (r_table[:, None, :] + re_table[None, :, :]).reshape(R * R, D)
    rr = jnp.pad(rr, ((0, 32 - R * R), (0, 0)))
    y, c = _stage1(e_table, wt, rr, wb, b_gv.reshape(1, D))

    uv2 = history_uv.astype(jnp.int32).reshape(B // PAIR, PROWS)
    ra = history_ra.astype(jnp.int32).reshape(-1)
    re = history_re.astype(jnp.int32).reshape(-1)
    neigh, selff = _stage2(y, uv2, ra, re, nodes.astype(jnp.int32),
                           feat_table, c)

    perm = jnp.concatenate([
        jnp.arange(0, 32, 2), jnp.arange(1, 32, 2),
        jnp.arange(32, 64, 2), jnp.arange(33, 64, 2)])
    w1b = W1[D:][perm, :]
    return _stage3(selff, neigh, W1[:D], w1b, b1.reshape(1, D))
